# baseline (speedup 1.0000x reference)
import functools

import jax
import jax.numpy as jnp
from jax import lax
from jax.experimental import pallas as pl
from jax.experimental.pallas import tpu as pltpu
from jax.experimental.pallas import tpu_sc as plsc

N, C, H, W = 2, 96, 512, 512
HW = H * W
NW = 32
PPW = HW // NW
B = 128
NCHUNK = PPW // B
NJ = C // 16
L = 16


def _floor_i32(v):
    t = v.astype(jnp.int32)
    tf = t.astype(jnp.float32)
    return jnp.where(tf > v, t - 1, t)


def _sc_warp_image(table, dxy, n_img):
    mesh = plsc.VectorSubcoreMesh(core_axis_name="c", subcore_axis_name="s")

    rows_t = pltpu.VMEM((B, C), jnp.float32)
    idx_t = pltpu.VMEM((B,), jnp.int32)
    w_t = pltpu.VMEM((B,), jnp.float32)

    @functools.partial(
        pl.kernel,
        mesh=mesh,
        compiler_params=pltpu.CompilerParams(use_tc_tiling_on_sc=False),
        out_type=jax.ShapeDtypeStruct((H, W, C), jnp.float32),
        scratch_types=[
            pltpu.VMEM((2, B), jnp.float32),
            pltpu.VMEM((2, B), jnp.float32),
            [idx_t] * 4,
            [idx_t] * 4,
            [w_t] * 4,
            [w_t] * 4,
            [rows_t] * 4,
            [rows_t] * 4,
            pltpu.SemaphoreType.DMA,
            pltpu.SemaphoreType.DMA,
            pltpu.SemaphoreType.DMA,
            pltpu.SemaphoreType.DMA,
        ],
    )
    def k(table_hbm, dxy_hbm, out_hbm,
          dxy0, dxy1, idx0, idx1, w0, w1, rows0, rows1,
          sg0, sg1, sw0, sw1):
        wid = lax.axis_index("s") * 2 + lax.axis_index("c")
        base0 = wid * PPW
        tab_n = table_hbm.at[n_img]
        gbase = n_img * HW
        bufs = ((dxy0, idx0, w0, rows0, sg0, sw0),
                (dxy1, idx1, w1, rows1, sg1, sw1))

        def out_slice(base):
            by = base // W
            bx = base % W
            return out_hbm.at[by, pl.ds(bx, B)]

        def stage(ci, buf):
            dxy_v, idx, w, rows, sg, _ = buf
            base = base0 + ci * B
            pltpu.sync_copy(dxy_hbm.at[:, pl.ds(gbase + base, B)], dxy_v)
            by = base // W
            bx = base % W
            byf = by.astype(jnp.float32)
            lanef = lax.iota(jnp.int32, L).astype(jnp.float32)
            for g in range(B // L):
                sl = pl.ds(g * L, L)
                ix = dxy_v[0, sl] + ((bx + g * L).astype(jnp.float32) + lanef)
                iy = dxy_v[1, sl] + byf
                x0 = _floor_i32(ix)
                y0 = _floor_i32(iy)
                wx1 = ix - x0.astype(jnp.float32)
                wx0 = 1.0 - wx1
                wy1 = iy - y0.astype(jnp.float32)
                wy0 = 1.0 - wy1
                x1 = x0 + 1
                y1 = y0 + 1
                zf = jnp.zeros((L,), jnp.float32)
                wx0m = jnp.where((x0 >= 0) & (x0 <= W - 1), wx0, zf)
                wx1m = jnp.where((x1 >= 0) & (x1 <= W - 1), wx1, zf)
                wy0m = jnp.where((y0 >= 0) & (y0 <= H - 1), wy0, zf)
                wy1m = jnp.where((y1 >= 0) & (y1 <= H - 1), wy1, zf)
                xc0 = jnp.clip(x0, 0, W - 1)
                xc1 = jnp.clip(x1, 0, W - 1)
                yc0 = jnp.clip(y0, 0, H - 1)
                yc1 = jnp.clip(y1, 0, H - 1)
                row0 = yc0 * W
                row1 = yc1 * W
                idx[0][sl] = row0 + xc0
                idx[1][sl] = row0 + xc1
                idx[2][sl] = row1 + xc0
                idx[3][sl] = row1 + xc1
                w[0][sl] = wy0m * wx0m
                w[1][sl] = wy0m * wx1m
                w[2][sl] = wy1m * wx0m
                w[3][sl] = wy1m * wx1m
            for q in range(4):
                pltpu.async_copy(tab_n.at[idx[q]], rows[q], sg)

        def drain(ci, buf):
            dxy_v, idx, w, rows, sg, sw = buf
            base = base0 + ci * B
            for q in range(4):
                pltpu.make_async_copy(tab_n.at[idx[q]], rows[q], sg).wait()

            def grp_body(g, carry2):
                sl2 = pl.ds(g * L, L)
                wv00 = w[0][sl2]
                wv01 = w[1][sl2]
                wv10 = w[2][sl2]
                wv11 = w[3][sl2]
                for bl in range(L):
                    b = g * L + bl
                    vw00 = wv00[bl]
                    vw01 = wv01[bl]
                    vw10 = wv10[bl]
                    vw11 = wv11[bl]
                    for j in range(NJ):
                        cs = pl.ds(j * L, L)
                        acc = (vw00 * rows[0][b, cs] + vw01 * rows[1][b, cs]
                               + vw10 * rows[2][b, cs] + vw11 * rows[3][b, cs])
                        rows[0][b, cs] = acc
                return carry2

            lax.fori_loop(0, B // L, grp_body, 0)
            pltpu.async_copy(rows[0], out_slice(base), sw)

        def wait_write(buf):
            rows, sw = buf[3], buf[5]
            pltpu.make_async_copy(rows[0], out_slice(base0), sw).wait()

        stage(0, bufs[0])

        def loop_body(kk, carry):
            for par in range(2):
                i = 2 * kk + par
                cur = bufs[par]
                nxt = bufs[1 - par]

                def do_stage():
                    @pl.when(i > 0)
                    def _():
                        wait_write(nxt)
                    stage(i + 1, nxt)

                pl.when(i + 1 < NCHUNK)(do_stage)
                drain(i, cur)
            return carry

        lax.fori_loop(0, NCHUNK // 2, loop_body, 0)
        wait_write(bufs[0])
        wait_write(bufs[1])

    return k(table, dxy)


def kernel(x, grid):
    table = jnp.transpose(x.reshape(N, C, HW), (0, 2, 1))
    dxy = jnp.stack((grid[..., 0].reshape(N * HW), grid[..., 1].reshape(N * HW)))
    outs = []
    for n in range(N):
        o = _sc_warp_image(table, dxy, n)
        outs.append(jnp.transpose(o, (2, 0, 1)))
    return jnp.stack(outs)


# speedup vs baseline: 1.6357x; 1.6357x over previous
import functools

import jax
import jax.numpy as jnp
from jax import lax
from jax.experimental import pallas as pl
from jax.experimental.pallas import tpu as pltpu
from jax.experimental.pallas import tpu_sc as plsc

N, C, H, W = 2, 96, 512, 512
HW = H * W
NW = 32
PPW = HW // NW
B = 128
NCHUNK = PPW // B
NJ = C // 16
L = 16


def _floor_i32(v):
    t = v.astype(jnp.int32)
    tf = t.astype(jnp.float32)
    return jnp.where(tf > v, t - 1, t)


def _sc_warp(table, dxy):
    mesh = plsc.VectorSubcoreMesh(core_axis_name="c", subcore_axis_name="s")

    rows_t = pltpu.VMEM((B, C), jnp.float32)
    idx_t = pltpu.VMEM((B,), jnp.int32)
    w_t = pltpu.VMEM((B,), jnp.float32)

    @functools.partial(
        pl.kernel,
        mesh=mesh,
        compiler_params=pltpu.CompilerParams(use_tc_tiling_on_sc=False),
        out_type=jax.ShapeDtypeStruct((N, HW, 128), jnp.float32),
        scratch_types=[
            pltpu.VMEM((2, B), jnp.float32),
            pltpu.VMEM((2, B), jnp.float32),
            [idx_t] * 4,
            [idx_t] * 4,
            [w_t] * 4,
            [w_t] * 4,
            [rows_t] * 4,
            [rows_t] * 4,
            pltpu.SemaphoreType.DMA,
            pltpu.SemaphoreType.DMA,
            pltpu.SemaphoreType.DMA,
            pltpu.SemaphoreType.DMA,
        ],
    )
    def k(table_hbm, dxy_hbm, out_hbm,
          dxy0, dxy1, idx0, idx1, w0, w1, rows0, rows1,
          sg0, sg1, sw0, sw1):
        wid = lax.axis_index("s") * 2 + lax.axis_index("c")
        base0 = wid * (N * HW // NW)
        n_img = base0 // HW
        tab_n = table_hbm.at[n_img]
        base0 = base0 - n_img * HW
        gbase = n_img * HW
        bufs = ((dxy0, idx0, w0, rows0, sg0, sw0),
                (dxy1, idx1, w1, rows1, sg1, sw1))

        def out_slice(base):
            return out_hbm.at[n_img, pl.ds(base, B), pl.ds(0, C)]

        def stage(ci, buf):
            dxy_v, idx, w, rows, sg, _ = buf
            base = base0 + ci * B
            pltpu.sync_copy(dxy_hbm.at[:, pl.ds(gbase + base, B)], dxy_v)
            by = base // W
            bx = base % W
            byf = by.astype(jnp.float32)
            lanef = lax.iota(jnp.int32, L).astype(jnp.float32)
            for g in range(B // L):
                sl = pl.ds(g * L, L)
                ix = dxy_v[0, sl] + ((bx + g * L).astype(jnp.float32) + lanef)
                iy = dxy_v[1, sl] + byf
                x0 = _floor_i32(ix)
                y0 = _floor_i32(iy)
                wx1 = ix - x0.astype(jnp.float32)
                wx0 = 1.0 - wx1
                wy1 = iy - y0.astype(jnp.float32)
                wy0 = 1.0 - wy1
                x1 = x0 + 1
                y1 = y0 + 1
                zf = jnp.zeros((L,), jnp.float32)
                wx0m = jnp.where((x0 >= 0) & (x0 <= W - 1), wx0, zf)
                wx1m = jnp.where((x1 >= 0) & (x1 <= W - 1), wx1, zf)
                wy0m = jnp.where((y0 >= 0) & (y0 <= H - 1), wy0, zf)
                wy1m = jnp.where((y1 >= 0) & (y1 <= H - 1), wy1, zf)
                xc0 = jnp.clip(x0, 0, W - 1)
                xc1 = jnp.clip(x1, 0, W - 1)
                yc0 = jnp.clip(y0, 0, H - 1)
                yc1 = jnp.clip(y1, 0, H - 1)
                row0 = yc0 * W
                row1 = yc1 * W
                idx[0][sl] = row0 + xc0
                idx[1][sl] = row0 + xc1
                idx[2][sl] = row1 + xc0
                idx[3][sl] = row1 + xc1
                w[0][sl] = wy0m * wx0m
                w[1][sl] = wy0m * wx1m
                w[2][sl] = wy1m * wx0m
                w[3][sl] = wy1m * wx1m
            for q in range(4):
                pltpu.async_copy(tab_n.at[idx[q]], rows[q], sg)

        def drain(ci, buf):
            dxy_v, idx, w, rows, sg, sw = buf
            base = base0 + ci * B
            for q in range(4):
                pltpu.make_async_copy(tab_n.at[idx[q]], rows[q], sg).wait()

            def grp_body(g, carry2):
                sl2 = pl.ds(g * L, L)
                wv00 = w[0][sl2]
                wv01 = w[1][sl2]
                wv10 = w[2][sl2]
                wv11 = w[3][sl2]
                for bl in range(L):
                    b = g * L + bl
                    vw00 = wv00[bl]
                    vw01 = wv01[bl]
                    vw10 = wv10[bl]
                    vw11 = wv11[bl]
                    for j in range(NJ):
                        cs = pl.ds(j * L, L)
                        acc = (vw00 * rows[0][b, cs] + vw01 * rows[1][b, cs]
                               + vw10 * rows[2][b, cs] + vw11 * rows[3][b, cs])
                        rows[0][b, cs] = acc
                return carry2

            lax.fori_loop(0, B // L, grp_body, 0)
            pltpu.async_copy(rows[0], out_slice(base), sw)

        def wait_write(buf):
            rows, sw = buf[3], buf[5]
            pltpu.make_async_copy(rows[0], out_slice(base0), sw).wait()

        stage(0, bufs[0])

        def loop_body(kk, carry):
            for par in range(2):
                i = 2 * kk + par
                cur = bufs[par]
                nxt = bufs[1 - par]

                def do_stage():
                    @pl.when(i > 0)
                    def _():
                        wait_write(nxt)
                    stage(i + 1, nxt)

                pl.when(i + 1 < NCHUNK)(do_stage)
                drain(i, cur)
            return carry

        lax.fori_loop(0, NCHUNK // 2, loop_body, 0)
        wait_write(bufs[0])
        wait_write(bufs[1])

    return k(table, dxy)


def kernel(x, grid):
    table = jnp.transpose(x.reshape(N, C, HW), (0, 2, 1))
    dxy = jnp.stack((grid[..., 0].reshape(N * HW), grid[..., 1].reshape(N * HW)))
    out = _sc_warp(table, dxy)
    return jnp.transpose(out[:, :, :C], (0, 2, 1)).reshape(N, C, H, W)
